# Initial kernel scaffold; baseline (speedup 1.0000x reference)
import jax
import jax.numpy as jnp
from jax.experimental import pallas as pl


def kernel(query, m_items):
    raise NotImplementedError("write your pallas kernel here")


# speedup vs baseline: 17.5953x; 17.5953x over previous
import functools

import jax
import jax.numpy as jnp
from jax import lax
from jax.experimental import pallas as pl


def _dot_nt(a, b):
    return lax.dot_general(a, b, (((1,), (1,)), ((), ())),
                           preferred_element_type=jnp.float32)


def _dot_nn(a, b):
    return lax.dot_general(a, b, (((1,), (0,)), ((), ())),
                           preferred_element_type=jnp.float32)


def _stats_kernel(q_ref, m_ref, cmax_ref, csum_ref, msum_ref, msq_ref):
    i = pl.program_id(0)
    m = m_ref[...]
    s = _dot_nt(q_ref[...], m)
    bmax = jnp.max(s, axis=0, keepdims=True)

    @pl.when(i == 0)
    def _():
        cmax_ref[...] = bmax
        csum_ref[...] = jnp.sum(jnp.exp(s - bmax), axis=0, keepdims=True)
        ones = jnp.ones((1, m.shape[1]), jnp.float32)
        msum_ref[...] = _dot_nt(ones, m)
        msq_ref[...] = _dot_nt(ones, m * m)

    @pl.when(i > 0)
    def _():
        old_max = cmax_ref[...]
        new_max = jnp.maximum(old_max, bmax)
        csum_ref[...] = (csum_ref[...] * jnp.exp(old_max - new_max)
                         + jnp.sum(jnp.exp(s - new_max), axis=0, keepdims=True))
        cmax_ref[...] = new_max


def _main_kernel(q_ref, m_ref, cmax_ref, csum_ref, msum_ref, msq_ref,
                 ssq_ref, ssm_ref, uq_ref, loss_ref):
    i = pl.program_id(0)
    bq, d = q_ref.shape
    mem = m_ref.shape[0]
    q = q_ref[...]
    m = m_ref[...]
    s = _dot_nt(q, m)
    rmax = jnp.max(s, axis=1, keepdims=True)
    e = jnp.exp(s - rmax)
    rsum = jnp.sum(e, axis=1, keepdims=True)
    inv_rsum = 1.0 / rsum
    ssm_ref[...] = e * inv_rsum
    ssq_ref[...] = jnp.exp(s - cmax_ref[...]) / csum_ref[...]
    uq = _dot_nn(e, m) * inv_rsum
    uq_ref[...] = uq
    recon = jnp.sum((q - uq) ** 2)

    iota = lax.broadcasted_iota(jnp.int32, (bq, mem), 1)
    msum_b = msum_ref[...]
    msq_b = msq_ref[...]
    work = e
    acc_u = jnp.zeros((bq, mem), jnp.float32)
    denom = jnp.zeros((bq, 1), jnp.float32)
    p1 = None
    stats = {}
    for k in range(6):
        v = jnp.max(work, axis=1, keepdims=True)
        eq = work >= v
        idx = jnp.min(jnp.where(eq, iota, mem), axis=1, keepdims=True)
        oh = iota == idx
        if k in (0, 5):
            s_top = jnp.sum(jnp.where(oh, s, 0.0), axis=1, keepdims=True)
            psum = jnp.sum(jnp.where(oh, msum_b, 0.0), axis=1, keepdims=True)
            psq = jnp.sum(jnp.where(oh, msq_b, 0.0), axis=1, keepdims=True)
            stats[k] = (s_top, psum, psq)
        if k == 0:
            p1 = v * inv_rsum
        if k < 5:
            w = jnp.exp(v * inv_rsum - p1)
            acc_u = acc_u + jnp.where(oh, w, 0.0)
            denom = denom + w
            work = jnp.where(oh, -1.0, work)

    gather = _dot_nn(acc_u, m) / denom
    gath = jnp.sum((gather - q) ** 2)

    qsq = jnp.sum(q * q, axis=1, keepdims=True)
    qsum = jnp.sum(q, axis=1, keepdims=True)
    eps = 1e-6
    ce = d * eps * eps
    s1, pm1, pq1 = stats[0]
    s6, pm6, pq6 = stats[5]
    d_ap = jnp.sqrt(qsq + pq1 - 2.0 * s1 + 2.0 * eps * (qsum - pm1) + ce)
    d_an = jnp.sqrt(qsq + pq6 - 2.0 * s6 + 2.0 * eps * (qsum - pm6) + ce)
    spread = jnp.sum(jnp.maximum(d_ap - d_an + 1.0, 0.0))

    li = lax.broadcasted_iota(jnp.int32, (1, 128), 1)
    contrib = (jnp.where(li == 0, recon, 0.0)
               + jnp.where(li == 1, gath, 0.0)
               + jnp.where(li == 2, spread, 0.0))

    @pl.when(i == 0)
    def _():
        loss_ref[...] = contrib

    @pl.when(i > 0)
    def _():
        loss_ref[...] = loss_ref[...] + contrib


@functools.partial(jax.jit, static_argnames=("interpret",))
def kernel(query, m_items, interpret=False):
    q_n, d = query.shape
    mem = m_items.shape[0]
    bq = min(128, q_n)
    nqb = q_n // bq

    f32 = jnp.float32
    stat_spec = pl.BlockSpec((1, mem), lambda i: (0, 0))
    stat_shape = jax.ShapeDtypeStruct((1, mem), f32)

    cmax, csum, msum, msq = pl.pallas_call(
        _stats_kernel,
        grid=(nqb,),
        in_specs=[pl.BlockSpec((bq, d), lambda i: (i, 0)),
                  pl.BlockSpec((mem, d), lambda i: (0, 0))],
        out_specs=[stat_spec] * 4,
        out_shape=[stat_shape] * 4,
        interpret=interpret,
    )(query, m_items)

    ssq, ssm, uq, loss = pl.pallas_call(
        _main_kernel,
        grid=(nqb,),
        in_specs=[pl.BlockSpec((bq, d), lambda i: (i, 0)),
                  pl.BlockSpec((mem, d), lambda i: (0, 0)),
                  stat_spec, stat_spec, stat_spec, stat_spec],
        out_specs=[pl.BlockSpec((bq, mem), lambda i: (i, 0)),
                   pl.BlockSpec((bq, mem), lambda i: (i, 0)),
                   pl.BlockSpec((bq, d), lambda i: (i, 0)),
                   pl.BlockSpec((1, 128), lambda i: (0, 0))],
        out_shape=[jax.ShapeDtypeStruct((q_n, mem), f32),
                   jax.ShapeDtypeStruct((q_n, mem), f32),
                   jax.ShapeDtypeStruct((q_n, d), f32),
                   jax.ShapeDtypeStruct((1, 128), f32)],
        interpret=interpret,
    )(query, m_items, cmax, csum, msum, msq)

    gathering_loss = loss[0, 1] / (q_n * d)
    spreading_loss = loss[0, 2] / q_n
    recon_loss = loss[0, 0] / (q_n * d)
    return (uq, m_items, ssq, ssm, gathering_loss, spreading_loss, recon_loss)


# speedup vs baseline: 20.2904x; 1.1532x over previous
import functools

import jax
import jax.numpy as jnp
from jax import lax
from jax.experimental import pallas as pl


def _dot_nt(a, b):
    return lax.dot_general(a, b, (((1,), (1,)), ((), ())),
                           preferred_element_type=jnp.float32)


def _dot_nn(a, b):
    return lax.dot_general(a, b, (((1,), (0,)), ((), ())),
                           preferred_element_type=jnp.float32)


def _stats_kernel(q_ref, m_ref, cmax_ref, cinv_ref):
    i = pl.program_id(0)
    n = pl.num_programs(0)
    s = _dot_nt(q_ref[...], m_ref[...])
    bmax = jnp.max(s, axis=0, keepdims=True)

    @pl.when(i == 0)
    def _():
        cmax_ref[...] = bmax
        cinv_ref[...] = jnp.sum(jnp.exp(s - bmax), axis=0, keepdims=True)

    @pl.when(i > 0)
    def _():
        old_max = cmax_ref[...]
        new_max = jnp.maximum(old_max, bmax)
        cinv_ref[...] = (cinv_ref[...] * jnp.exp(old_max - new_max)
                         + jnp.sum(jnp.exp(s - new_max), axis=0, keepdims=True))
        cmax_ref[...] = new_max

    @pl.when(i == n - 1)
    def _():
        cinv_ref[...] = 1.0 / cinv_ref[...]


def _main_kernel(q_ref, m_ref, cmax_ref, cinv_ref,
                 ssq_ref, ssm_ref, uq_ref, loss_ref):
    i = pl.program_id(0)
    bq, d = q_ref.shape
    mem = m_ref.shape[0]
    q = q_ref[...]
    m = m_ref[...]
    s = _dot_nt(q, m)
    rmax = jnp.max(s, axis=1, keepdims=True)
    e = jnp.exp(s - rmax)
    rsum = jnp.sum(e, axis=1, keepdims=True)
    inv_rsum = 1.0 / rsum
    ssm_ref[...] = e * inv_rsum
    ssq_ref[...] = jnp.exp(s - cmax_ref[...]) * cinv_ref[...]
    uq = _dot_nn(e, m) * inv_rsum
    uq_ref[...] = uq
    recon = jnp.sum((q - uq) ** 2)

    iota = lax.broadcasted_iota(jnp.int32, (bq, mem), 1)
    work = e
    denom = jnp.zeros((bq, 1), jnp.float32)
    p1 = None
    ohf = {}
    s_top = {}
    for k in range(6):
        idx = jnp.argmax(work, axis=1).reshape(bq, 1)
        v = jnp.max(work, axis=1, keepdims=True)
        oh = iota == idx
        if k in (0, 5):
            ohf[k] = jnp.where(oh, 1.0, 0.0)
            s_top[k] = jnp.sum(jnp.where(oh, s, 0.0), axis=1, keepdims=True)
        if k == 0:
            p1 = v * inv_rsum
        if k < 5:
            w = jnp.exp(v * inv_rsum - p1)
            denom = denom + w
            work = jnp.where(oh, -w - 2.0, work)

    u = jnp.maximum(-work - 2.0, 0.0)
    wcat = jnp.concatenate([u, ohf[0], ohf[5]], axis=0)
    g3 = _dot_nn(wcat, m)
    gather = g3[:bq] / denom
    pos = g3[bq:2 * bq]
    neg = g3[2 * bq:]
    gath = jnp.sum((gather - q) ** 2)

    eps = 1e-6
    d_ap = jnp.sqrt(jnp.sum((q - pos + eps) ** 2, axis=1, keepdims=True))
    d_an = jnp.sqrt(jnp.sum((q - neg + eps) ** 2, axis=1, keepdims=True))
    spread = jnp.sum(jnp.maximum(d_ap - d_an + 1.0, 0.0))

    li = lax.broadcasted_iota(jnp.int32, (1, 128), 1)
    contrib = (jnp.where(li == 0, recon, 0.0)
               + jnp.where(li == 1, gath, 0.0)
               + jnp.where(li == 2, spread, 0.0))

    @pl.when(i == 0)
    def _():
        loss_ref[...] = contrib

    @pl.when(i > 0)
    def _():
        loss_ref[...] = loss_ref[...] + contrib


@functools.partial(jax.jit, static_argnames=("interpret",))
def kernel(query, m_items, interpret=False):
    q_n, d = query.shape
    mem = m_items.shape[0]
    bq = min(128, q_n)
    nqb = q_n // bq

    f32 = jnp.float32
    stat_spec = pl.BlockSpec((1, mem), lambda i: (0, 0))
    stat_shape = jax.ShapeDtypeStruct((1, mem), f32)

    cmax, cinv = pl.pallas_call(
        _stats_kernel,
        grid=(nqb,),
        in_specs=[pl.BlockSpec((bq, d), lambda i: (i, 0)),
                  pl.BlockSpec((mem, d), lambda i: (0, 0))],
        out_specs=[stat_spec] * 2,
        out_shape=[stat_shape] * 2,
        interpret=interpret,
    )(query, m_items)

    ssq, ssm, uq, loss = pl.pallas_call(
        _main_kernel,
        grid=(nqb,),
        in_specs=[pl.BlockSpec((bq, d), lambda i: (i, 0)),
                  pl.BlockSpec((mem, d), lambda i: (0, 0)),
                  stat_spec, stat_spec],
        out_specs=[pl.BlockSpec((bq, mem), lambda i: (i, 0)),
                   pl.BlockSpec((bq, mem), lambda i: (i, 0)),
                   pl.BlockSpec((bq, d), lambda i: (i, 0)),
                   pl.BlockSpec((1, 128), lambda i: (0, 0))],
        out_shape=[jax.ShapeDtypeStruct((q_n, mem), f32),
                   jax.ShapeDtypeStruct((q_n, mem), f32),
                   jax.ShapeDtypeStruct((q_n, d), f32),
                   jax.ShapeDtypeStruct((1, 128), f32)],
        interpret=interpret,
    )(query, m_items, cmax, cinv)

    gathering_loss = loss[0, 1] / (q_n * d)
    spreading_loss = loss[0, 2] / q_n
    recon_loss = loss[0, 0] / (q_n * d)
    return (uq, m_items, ssq, ssm, gathering_loss, spreading_loss, recon_loss)


# speedup vs baseline: 26.3832x; 1.3003x over previous
import functools

import jax
import jax.numpy as jnp
from jax import lax
from jax.experimental import pallas as pl
from jax.experimental.pallas import tpu as pltpu


def _dot_nt(a, b):
    return lax.dot_general(a, b, (((1,), (1,)), ((), ())),
                           preferred_element_type=jnp.float32)


def _dot_nn(a, b):
    return lax.dot_general(a, b, (((1,), (0,)), ((), ())),
                           preferred_element_type=jnp.float32)


def _stats_kernel(q_ref, m_ref, cmax_ref, cinv_ref):
    i = pl.program_id(0)
    n = pl.num_programs(0)
    s = _dot_nt(q_ref[...], m_ref[...])
    bmax = jnp.max(s, axis=0, keepdims=True)

    @pl.when(i == 0)
    def _():
        cmax_ref[...] = bmax
        cinv_ref[...] = jnp.sum(jnp.exp(s - bmax), axis=0, keepdims=True)

    @pl.when(i > 0)
    def _():
        old_max = cmax_ref[...]
        new_max = jnp.maximum(old_max, bmax)
        cinv_ref[...] = (cinv_ref[...] * jnp.exp(old_max - new_max)
                         + jnp.sum(jnp.exp(s - new_max), axis=0, keepdims=True))
        cmax_ref[...] = new_max

    @pl.when(i == n - 1)
    def _():
        cinv_ref[...] = 1.0 / cinv_ref[...]


def _main_kernel(q_ref, m_ref, cmax_ref, cinv_ref,
                 ssq_ref, ssm_ref, uq_ref, loss_ref,
                 g_ref, pos_ref, neg_ref):
    i = pl.program_id(0)
    bq, d = q_ref.shape
    mem = m_ref.shape[0]
    q = q_ref[...]
    m = m_ref[...]
    s = _dot_nt(q, m)
    rmax = jnp.max(s, axis=1, keepdims=True)
    e = jnp.exp(s - rmax)
    rsum = jnp.sum(e, axis=1, keepdims=True)
    inv_rsum = 1.0 / rsum
    ssm_ref[...] = e * inv_rsum
    ssq_ref[...] = jnp.exp(s - cmax_ref[...]) * cinv_ref[...]
    uq = _dot_nn(e, m) * inv_rsum
    uq_ref[...] = uq
    recon = jnp.sum((q - uq) ** 2)

    def finish(work, ohf0, ohf5):
        u = jnp.maximum(-work - 2.0, 0.0)
        denom = jnp.sum(u, axis=1, keepdims=True)
        g_ref[...] = _dot_nn(u, m) / denom
        pos_ref[...] = _dot_nn(ohf0, m)
        neg_ref[...] = _dot_nn(ohf5, m)

    work = ssm_ref[...]
    p1 = jnp.max(work, axis=1, keepdims=True)
    v = p1
    ohf0 = ohf5 = None
    for k in range(6):
        if k > 0:
            v = jnp.max(work, axis=1, keepdims=True)
        eq = work == v
        if k == 0:
            ohf0 = jnp.where(eq, 1.0, 0.0)
        if k == 5:
            ohf5 = jnp.where(eq, 1.0, 0.0)
        if k < 5:
            w = jnp.exp(v - p1)
            work = jnp.where(eq, -w - 2.0, work)

    n_masked = jnp.sum(jnp.where(work < -1.5, 1.0, 0.0))
    n_top6 = jnp.sum(ohf5)
    ties = (n_masked != 5.0 * bq) | (n_top6 != 1.0 * bq)
    finish(work, ohf0, ohf5)

    @pl.when(ties)
    def _():
        iota = lax.broadcasted_iota(jnp.int32, (bq, mem), 1)
        wk = ssm_ref[...]
        of0 = of5 = None
        for k in range(6):
            idx = jnp.argmax(wk, axis=1).reshape(bq, 1)
            oh = iota == idx
            if k == 0:
                of0 = jnp.where(oh, 1.0, 0.0)
            if k == 5:
                of5 = jnp.where(oh, 1.0, 0.0)
            if k < 5:
                vv = jnp.max(wk, axis=1, keepdims=True)
                ww = jnp.exp(vv - p1)
                wk = jnp.where(oh, -ww - 2.0, wk)
        finish(wk, of0, of5)

    gather = g_ref[...]
    pos = pos_ref[...]
    neg = neg_ref[...]
    gath = jnp.sum((gather - q) ** 2)

    eps = 1e-6
    d_ap = jnp.sqrt(jnp.sum((q - pos + eps) ** 2, axis=1, keepdims=True))
    d_an = jnp.sqrt(jnp.sum((q - neg + eps) ** 2, axis=1, keepdims=True))
    spread = jnp.sum(jnp.maximum(d_ap - d_an + 1.0, 0.0))

    li = lax.broadcasted_iota(jnp.int32, (1, 128), 1)
    contrib = (jnp.where(li == 0, recon, 0.0)
               + jnp.where(li == 1, gath, 0.0)
               + jnp.where(li == 2, spread, 0.0))

    @pl.when(i == 0)
    def _():
        loss_ref[...] = contrib

    @pl.when(i > 0)
    def _():
        loss_ref[...] = loss_ref[...] + contrib


@functools.partial(jax.jit, static_argnames=("interpret",))
def kernel(query, m_items, interpret=False):
    q_n, d = query.shape
    mem = m_items.shape[0]
    bq = min(128, q_n)
    nqb = q_n // bq

    f32 = jnp.float32
    stat_spec = pl.BlockSpec((1, mem), lambda i: (0, 0))
    stat_shape = jax.ShapeDtypeStruct((1, mem), f32)

    cmax, cinv = pl.pallas_call(
        _stats_kernel,
        grid=(nqb,),
        in_specs=[pl.BlockSpec((bq, d), lambda i: (i, 0)),
                  pl.BlockSpec((mem, d), lambda i: (0, 0))],
        out_specs=[stat_spec] * 2,
        out_shape=[stat_shape] * 2,
        interpret=interpret,
    )(query, m_items)

    ssq, ssm, uq, loss = pl.pallas_call(
        _main_kernel,
        grid=(nqb,),
        in_specs=[pl.BlockSpec((bq, d), lambda i: (i, 0)),
                  pl.BlockSpec((mem, d), lambda i: (0, 0)),
                  stat_spec, stat_spec],
        out_specs=[pl.BlockSpec((bq, mem), lambda i: (i, 0)),
                   pl.BlockSpec((bq, mem), lambda i: (i, 0)),
                   pl.BlockSpec((bq, d), lambda i: (i, 0)),
                   pl.BlockSpec((1, 128), lambda i: (0, 0))],
        out_shape=[jax.ShapeDtypeStruct((q_n, mem), f32),
                   jax.ShapeDtypeStruct((q_n, mem), f32),
                   jax.ShapeDtypeStruct((q_n, d), f32),
                   jax.ShapeDtypeStruct((1, 128), f32)],
        scratch_shapes=[pltpu.VMEM((bq, d), f32),
                        pltpu.VMEM((bq, d), f32),
                        pltpu.VMEM((bq, d), f32)],
        compiler_params=pltpu.CompilerParams(vmem_limit_bytes=100 * 1024 * 1024),
        interpret=interpret,
    )(query, m_items, cmax, cinv)

    gathering_loss = loss[0, 1] / (q_n * d)
    spreading_loss = loss[0, 2] / q_n
    recon_loss = loss[0, 0] / (q_n * d)
    return (uq, m_items, ssq, ssm, gathering_loss, spreading_loss, recon_loss)


# speedup vs baseline: 28.8112x; 1.0920x over previous
import functools

import jax
import jax.numpy as jnp
from jax import lax
from jax.experimental import pallas as pl
from jax.experimental.pallas import tpu as pltpu


def _dot_nt(a, b):
    return lax.dot_general(a, b, (((1,), (1,)), ((), ())),
                           preferred_element_type=jnp.float32)


def _dot_nn(a, b):
    return lax.dot_general(a, b, (((1,), (0,)), ((), ())),
                           preferred_element_type=jnp.float32)


def _stats_kernel(q_ref, m_ref, cmax_ref, cinv_ref):
    i = pl.program_id(0)
    n = pl.num_programs(0)
    s = _dot_nt(q_ref[...], m_ref[...])
    bmax = jnp.max(s, axis=0, keepdims=True)

    @pl.when(i == 0)
    def _():
        cmax_ref[...] = bmax
        cinv_ref[...] = jnp.sum(jnp.exp(s - bmax), axis=0, keepdims=True)

    @pl.when(i > 0)
    def _():
        old_max = cmax_ref[...]
        new_max = jnp.maximum(old_max, bmax)
        cinv_ref[...] = (cinv_ref[...] * jnp.exp(old_max - new_max)
                         + jnp.sum(jnp.exp(s - new_max), axis=0, keepdims=True))
        cmax_ref[...] = new_max

    @pl.when(i == n - 1)
    def _():
        cinv_ref[...] = 1.0 / cinv_ref[...]


def _main_kernel(q_ref, m_ref, cmax_ref, cinv_ref,
                 ssq_ref, ssm_ref, uq_ref, loss_ref,
                 g_ref, pos_ref, neg_ref):
    i = pl.program_id(0)
    bq, d = q_ref.shape
    mem = m_ref.shape[0]
    q = q_ref[...]
    m = m_ref[...]
    s = _dot_nt(q, m)
    rmax = jnp.max(s, axis=1, keepdims=True)
    e = jnp.exp(s - rmax)
    rsum = jnp.sum(e, axis=1, keepdims=True)
    inv_rsum = 1.0 / rsum
    ssm_ref[...] = e * inv_rsum
    ssq_ref[...] = jnp.exp(s - cmax_ref[...]) * cinv_ref[...]
    uq = _dot_nn(e, m) * inv_rsum
    uq_ref[...] = uq
    recon = jnp.sum((q - uq) ** 2)

    def finish(work, ohf0, ohf5):
        u = jnp.maximum(-work - 2.0, 0.0)
        denom = jnp.sum(u, axis=1, keepdims=True)
        g_ref[...] = _dot_nn(u, m) / denom
        pos_ref[...] = _dot_nn(ohf0, m)
        neg_ref[...] = _dot_nn(ohf5, m)

    work = ssm_ref[...]
    p1 = jnp.max(work, axis=1, keepdims=True)
    v = p1
    ohf0 = ohf5 = None
    for k in range(6):
        if k > 0:
            v = jnp.max(work, axis=1, keepdims=True)
        eq = work == v
        if k == 0:
            ohf0 = jnp.where(eq, 1.0, 0.0)
        if k == 5:
            ohf5 = jnp.where(eq, 1.0, 0.0)
        if k < 5:
            w = jnp.exp(v - p1)
            work = jnp.where(eq, -w - 2.0, work)

    n_masked = jnp.sum(jnp.where(work < -1.5, 1.0, 0.0))
    n_top6 = jnp.sum(ohf5)
    ties = (n_masked != 5.0 * bq) | (n_top6 != 1.0 * bq)
    finish(work, ohf0, ohf5)

    @pl.when(ties)
    def _():
        iota = lax.broadcasted_iota(jnp.int32, (bq, mem), 1)
        wk = ssm_ref[...]
        of0 = of5 = None
        for k in range(6):
            idx = jnp.argmax(wk, axis=1).reshape(bq, 1)
            oh = iota == idx
            if k == 0:
                of0 = jnp.where(oh, 1.0, 0.0)
            if k == 5:
                of5 = jnp.where(oh, 1.0, 0.0)
            if k < 5:
                vv = jnp.max(wk, axis=1, keepdims=True)
                ww = jnp.exp(vv - p1)
                wk = jnp.where(oh, -ww - 2.0, wk)
        finish(wk, of0, of5)

    gather = g_ref[...]
    pos = pos_ref[...]
    neg = neg_ref[...]
    gath = jnp.sum((gather - q) ** 2)

    eps = 1e-6
    d_ap = jnp.sqrt(jnp.sum((q - pos + eps) ** 2, axis=1, keepdims=True))
    d_an = jnp.sqrt(jnp.sum((q - neg + eps) ** 2, axis=1, keepdims=True))
    spread = jnp.sum(jnp.maximum(d_ap - d_an + 1.0, 0.0))

    li = lax.broadcasted_iota(jnp.int32, (1, 128), 1)
    contrib = (jnp.where(li == 0, recon, 0.0)
               + jnp.where(li == 1, gath, 0.0)
               + jnp.where(li == 2, spread, 0.0))

    @pl.when(i == 0)
    def _():
        loss_ref[...] = contrib

    @pl.when(i > 0)
    def _():
        loss_ref[...] = loss_ref[...] + contrib


@functools.partial(jax.jit, static_argnames=("interpret",))
def kernel(query, m_items, interpret=False):
    q_n, d = query.shape
    mem = m_items.shape[0]
    bq = min(128, q_n)
    nqb = q_n // bq
    bqs = min(256, q_n)

    f32 = jnp.float32
    stat_spec = pl.BlockSpec((1, mem), lambda i: (0, 0))
    stat_shape = jax.ShapeDtypeStruct((1, mem), f32)

    cmax, cinv = pl.pallas_call(
        _stats_kernel,
        grid=(q_n // bqs,),
        in_specs=[pl.BlockSpec((bqs, d), lambda i: (i, 0)),
                  pl.BlockSpec((mem, d), lambda i: (0, 0))],
        out_specs=[stat_spec] * 2,
        out_shape=[stat_shape] * 2,
        interpret=interpret,
    )(query, m_items)

    ssq, ssm, uq, loss = pl.pallas_call(
        _main_kernel,
        grid=(nqb,),
        in_specs=[pl.BlockSpec((bq, d), lambda i: (i, 0)),
                  pl.BlockSpec((mem, d), lambda i: (0, 0)),
                  stat_spec, stat_spec],
        out_specs=[pl.BlockSpec((bq, mem), lambda i: (i, 0)),
                   pl.BlockSpec((bq, mem), lambda i: (i, 0)),
                   pl.BlockSpec((bq, d), lambda i: (i, 0)),
                   pl.BlockSpec((1, 128), lambda i: (0, 0))],
        out_shape=[jax.ShapeDtypeStruct((q_n, mem), f32),
                   jax.ShapeDtypeStruct((q_n, mem), f32),
                   jax.ShapeDtypeStruct((q_n, d), f32),
                   jax.ShapeDtypeStruct((1, 128), f32)],
        scratch_shapes=[pltpu.VMEM((bq, d), f32),
                        pltpu.VMEM((bq, d), f32),
                        pltpu.VMEM((bq, d), f32)],
        compiler_params=pltpu.CompilerParams(vmem_limit_bytes=100 * 1024 * 1024),
        interpret=interpret,
    )(query, m_items, cmax, cinv)

    gathering_loss = loss[0, 1] / (q_n * d)
    spreading_loss = loss[0, 2] / q_n
    recon_loss = loss[0, 0] / (q_n * d)
    return (uq, m_items, ssq, ssm, gathering_loss, spreading_loss, recon_loss)


# speedup vs baseline: 32.1101x; 1.1145x over previous
import functools

import jax
import jax.numpy as jnp
from jax import lax
from jax.experimental import pallas as pl
from jax.experimental.pallas import tpu as pltpu


def _dot_nt(a, b):
    return lax.dot_general(a, b, (((1,), (1,)), ((), ())),
                           preferred_element_type=jnp.float32)


def _dot_nn(a, b):
    return lax.dot_general(a, b, (((1,), (0,)), ((), ())),
                           preferred_element_type=jnp.float32)


def _stats_kernel(q_ref, m_ref, cmax_ref, cinv_ref):
    i = pl.program_id(0)
    n = pl.num_programs(0)
    s = _dot_nt(q_ref[...], m_ref[...])
    bmax = jnp.max(s, axis=0, keepdims=True)

    @pl.when(i == 0)
    def _():
        cmax_ref[...] = bmax
        cinv_ref[...] = jnp.sum(jnp.exp(s - bmax), axis=0, keepdims=True)

    @pl.when(i > 0)
    def _():
        old_max = cmax_ref[...]
        new_max = jnp.maximum(old_max, bmax)
        cinv_ref[...] = (cinv_ref[...] * jnp.exp(old_max - new_max)
                         + jnp.sum(jnp.exp(s - new_max), axis=0, keepdims=True))
        cmax_ref[...] = new_max

    @pl.when(i == n - 1)
    def _():
        cinv_ref[...] = 1.0 / cinv_ref[...]


def _tail_losses(q, m, gather, pos, neg):
    gath = jnp.sum((gather - q) ** 2)
    eps = 1e-6
    d_ap = jnp.sqrt(jnp.sum((q - pos + eps) ** 2, axis=1, keepdims=True))
    d_an = jnp.sqrt(jnp.sum((q - neg + eps) ** 2, axis=1, keepdims=True))
    spread = jnp.sum(jnp.maximum(d_ap - d_an + 1.0, 0.0))
    return gath, spread


def _combine(work, ohf0, ohf5, m):
    u = jnp.maximum(-work - 2.0, 0.0)
    denom = jnp.sum(u, axis=1, keepdims=True)
    gather = _dot_nn(u, m) / denom
    pos = _dot_nn(ohf0, m)
    neg = _dot_nn(ohf5, m)
    return gather, pos, neg


def _main_kernel(q_ref, m_ref, cmax_ref, cinv_ref,
                 ssq_ref, ssm_ref, uq_ref, loss_ref):
    i = pl.program_id(0)
    bq, d = q_ref.shape
    q = q_ref[...]
    m = m_ref[...]
    s = _dot_nt(q, m)
    rmax = jnp.max(s, axis=1, keepdims=True)
    e = jnp.exp(s - rmax)
    rsum = jnp.sum(e, axis=1, keepdims=True)
    inv_rsum = 1.0 / rsum
    ssm_ref[...] = e * inv_rsum
    ssq_ref[...] = jnp.exp(s - cmax_ref[...]) * cinv_ref[...]
    uq = _dot_nn(e, m) * inv_rsum
    uq_ref[...] = uq
    recon = jnp.sum((q - uq) ** 2)

    work = ssm_ref[...]
    p1 = jnp.max(work, axis=1, keepdims=True)
    v = p1
    pos = neg = n_top6 = None
    for k in range(6):
        if k > 0:
            v = jnp.max(work, axis=1, keepdims=True)
        eq = work == v
        if k == 0:
            pos = _dot_nn(jnp.where(eq, 1.0, 0.0), m)
        if k == 5:
            eqf = jnp.where(eq, 1.0, 0.0)
            n_top6 = jnp.sum(eqf)
            neg = _dot_nn(eqf, m)
        if k < 5:
            w = jnp.exp(v - p1)
            work = jnp.where(eq, -w - 2.0, work)

    n_masked = jnp.sum(jnp.where(work < -1.5, 1.0, 0.0))
    tie = jnp.where((n_masked != 5.0 * bq) | (n_top6 != 1.0 * bq), 1.0, 0.0)

    u = jnp.maximum(-work - 2.0, 0.0)
    denom = jnp.sum(u, axis=1, keepdims=True)
    gather = _dot_nn(u, m) / denom
    gath, spread = _tail_losses(q, m, gather, pos, neg)

    li = lax.broadcasted_iota(jnp.int32, (1, 128), 1)
    contrib = (jnp.where(li == 0, recon, 0.0)
               + jnp.where(li == 1, gath, 0.0)
               + jnp.where(li == 2, spread, 0.0)
               + jnp.where(li == 3, tie, 0.0))

    @pl.when(i == 0)
    def _():
        loss_ref[...] = contrib

    @pl.when(i > 0)
    def _():
        loss_ref[...] = loss_ref[...] + contrib


def _exact_kernel(q_ref, m_ref, ssm_ref, loss_ref):
    i = pl.program_id(0)
    bq, d = q_ref.shape
    q = q_ref[...]
    m = m_ref[...]
    p = ssm_ref[...]
    p1 = jnp.max(p, axis=1, keepdims=True)
    iota = lax.broadcasted_iota(jnp.int32, p.shape, 1)
    wk = p
    ohf0 = ohf5 = None
    for k in range(6):
        idx = jnp.argmax(wk, axis=1).reshape(bq, 1)
        oh = iota == idx
        if k == 0:
            ohf0 = jnp.where(oh, 1.0, 0.0)
        if k == 5:
            ohf5 = jnp.where(oh, 1.0, 0.0)
        if k < 5:
            vv = jnp.max(wk, axis=1, keepdims=True)
            ww = jnp.exp(vv - p1)
            wk = jnp.where(oh, -ww - 2.0, wk)

    gather, pos, neg = _combine(wk, ohf0, ohf5, m)
    gath, spread = _tail_losses(q, m, gather, pos, neg)

    li = lax.broadcasted_iota(jnp.int32, (1, 128), 1)
    contrib = (jnp.where(li == 0, gath, 0.0)
               + jnp.where(li == 1, spread, 0.0))

    @pl.when(i == 0)
    def _():
        loss_ref[...] = contrib

    @pl.when(i > 0)
    def _():
        loss_ref[...] = loss_ref[...] + contrib


@functools.partial(jax.jit, static_argnames=("interpret",))
def kernel(query, m_items, interpret=False):
    q_n, d = query.shape
    mem = m_items.shape[0]
    bq = min(256, q_n)
    nqb = q_n // bq

    f32 = jnp.float32
    stat_spec = pl.BlockSpec((1, mem), lambda i: (0, 0))
    stat_shape = jax.ShapeDtypeStruct((1, mem), f32)

    cmax, cinv = pl.pallas_call(
        _stats_kernel,
        grid=(nqb,),
        in_specs=[pl.BlockSpec((bq, d), lambda i: (i, 0)),
                  pl.BlockSpec((mem, d), lambda i: (0, 0))],
        out_specs=[stat_spec] * 2,
        out_shape=[stat_shape] * 2,
        interpret=interpret,
    )(query, m_items)

    ssq, ssm, uq, loss = pl.pallas_call(
        _main_kernel,
        grid=(nqb,),
        in_specs=[pl.BlockSpec((bq, d), lambda i: (i, 0)),
                  pl.BlockSpec((mem, d), lambda i: (0, 0)),
                  stat_spec, stat_spec],
        out_specs=[pl.BlockSpec((bq, mem), lambda i: (i, 0)),
                   pl.BlockSpec((bq, mem), lambda i: (i, 0)),
                   pl.BlockSpec((bq, d), lambda i: (i, 0)),
                   pl.BlockSpec((1, 128), lambda i: (0, 0))],
        out_shape=[jax.ShapeDtypeStruct((q_n, mem), f32),
                   jax.ShapeDtypeStruct((q_n, mem), f32),
                   jax.ShapeDtypeStruct((q_n, d), f32),
                   jax.ShapeDtypeStruct((1, 128), f32)],
        compiler_params=pltpu.CompilerParams(vmem_limit_bytes=100 * 1024 * 1024),
        interpret=interpret,
    )(query, m_items, cmax, cinv)

    def exact_tail():
        bqe = min(128, q_n)
        loss2 = pl.pallas_call(
            _exact_kernel,
            grid=(q_n // bqe,),
            in_specs=[pl.BlockSpec((bqe, d), lambda i: (i, 0)),
                      pl.BlockSpec((mem, d), lambda i: (0, 0)),
                      pl.BlockSpec((bqe, mem), lambda i: (i, 0))],
            out_specs=pl.BlockSpec((1, 128), lambda i: (0, 0)),
            out_shape=jax.ShapeDtypeStruct((1, 128), f32),
            compiler_params=pltpu.CompilerParams(
                vmem_limit_bytes=100 * 1024 * 1024),
            interpret=interpret,
        )(query, m_items, ssm)
        return loss2[0, 0], loss2[0, 1]

    gath_sum, spread_sum = lax.cond(
        loss[0, 3] > 0.0,
        exact_tail,
        lambda: (loss[0, 1], loss[0, 2]))

    gathering_loss = gath_sum / (q_n * d)
    spreading_loss = spread_sum / q_n
    recon_loss = loss[0, 0] / (q_n * d)
    return (uq, m_items, ssq, ssm, gathering_loss, spreading_loss, recon_loss)


# speedup vs baseline: 32.3235x; 1.0066x over previous
import functools

import jax
import jax.numpy as jnp
from jax import lax
from jax.experimental import pallas as pl
from jax.experimental.pallas import tpu as pltpu


def _dot_nt(a, b):
    return lax.dot_general(a, b, (((1,), (1,)), ((), ())),
                           preferred_element_type=jnp.float32)


def _dot_nn(a, b):
    return lax.dot_general(a, b, (((1,), (0,)), ((), ())),
                           preferred_element_type=jnp.float32)


def _stats_kernel(q_ref, m_ref, cmax_ref, cinv_ref):
    i = pl.program_id(0)
    n = pl.num_programs(0)
    s = _dot_nt(q_ref[...], m_ref[...])
    bmax = jnp.max(s, axis=0, keepdims=True)

    @pl.when(i == 0)
    def _():
        cmax_ref[...] = bmax
        cinv_ref[...] = jnp.sum(jnp.exp(s - bmax), axis=0, keepdims=True)

    @pl.when(i > 0)
    def _():
        old_max = cmax_ref[...]
        new_max = jnp.maximum(old_max, bmax)
        cinv_ref[...] = (cinv_ref[...] * jnp.exp(old_max - new_max)
                         + jnp.sum(jnp.exp(s - new_max), axis=0, keepdims=True))
        cmax_ref[...] = new_max

    @pl.when(i == n - 1)
    def _():
        cinv_ref[...] = 1.0 / cinv_ref[...]


def _tail_losses(q, m, gather, pos, neg):
    gath = jnp.sum((gather - q) ** 2)
    eps = 1e-6
    d_ap = jnp.sqrt(jnp.sum((q - pos + eps) ** 2, axis=1, keepdims=True))
    d_an = jnp.sqrt(jnp.sum((q - neg + eps) ** 2, axis=1, keepdims=True))
    spread = jnp.sum(jnp.maximum(d_ap - d_an + 1.0, 0.0))
    return gath, spread


def _combine(work, ohf0, ohf5, m):
    u = jnp.maximum(-work - 2.0, 0.0)
    denom = jnp.sum(u, axis=1, keepdims=True)
    gather = _dot_nn(u, m) / denom
    pos = _dot_nn(ohf0, m)
    neg = _dot_nn(ohf5, m)
    return gather, pos, neg


def _main_kernel(q_ref, m_ref, cmax_ref, cinv_ref,
                 ssq_ref, ssm_ref, uq_ref, loss_ref):
    i = pl.program_id(0)
    bq, d = q_ref.shape
    q = q_ref[...]
    m = m_ref[...]
    s = _dot_nt(q, m)
    rmax = jnp.max(s, axis=1, keepdims=True)
    e = jnp.exp(s - rmax)
    rsum = jnp.sum(e, axis=1, keepdims=True)
    inv_rsum = 1.0 / rsum
    ssm_ref[...] = e * inv_rsum
    ssq_ref[...] = jnp.exp(s - cmax_ref[...]) * cinv_ref[...]
    uq = _dot_nn(e, m) * inv_rsum
    uq_ref[...] = uq
    recon = jnp.sum((q - uq) ** 2)

    work = ssm_ref[...]
    p1 = jnp.max(work, axis=1, keepdims=True)
    v = p1
    pos = neg = n_top6 = None
    for k in range(6):
        if k > 0:
            v = jnp.max(work, axis=1, keepdims=True)
        eq = work == v
        if k == 0:
            pos = _dot_nn(jnp.where(eq, 1.0, 0.0), m)
        if k == 5:
            eqf = jnp.where(eq, 1.0, 0.0)
            n_top6 = jnp.sum(eqf)
            neg = _dot_nn(eqf, m)
        if k < 5:
            w = jnp.exp(v - p1)
            work = jnp.where(eq, -w - 2.0, work)

    n_masked = jnp.sum(jnp.where(work < -1.5, 1.0, 0.0))
    tie = jnp.where((n_masked != 5.0 * bq) | (n_top6 != 1.0 * bq), 1.0, 0.0)

    u = jnp.maximum(-work - 2.0, 0.0)
    denom = jnp.sum(u, axis=1, keepdims=True)
    gather = _dot_nn(u, m) / denom
    gath, spread = _tail_losses(q, m, gather, pos, neg)

    li = lax.broadcasted_iota(jnp.int32, (1, 128), 1)
    contrib = (jnp.where(li == 0, recon, 0.0)
               + jnp.where(li == 1, gath, 0.0)
               + jnp.where(li == 2, spread, 0.0)
               + jnp.where(li == 3, tie, 0.0))

    @pl.when(i == 0)
    def _():
        loss_ref[...] = contrib

    @pl.when(i > 0)
    def _():
        loss_ref[...] = loss_ref[...] + contrib


def _exact_kernel(q_ref, m_ref, ssm_ref, loss_ref):
    i = pl.program_id(0)
    bq, d = q_ref.shape
    q = q_ref[...]
    m = m_ref[...]
    p = ssm_ref[...]
    p1 = jnp.max(p, axis=1, keepdims=True)
    iota = lax.broadcasted_iota(jnp.int32, p.shape, 1)
    wk = p
    ohf0 = ohf5 = None
    for k in range(6):
        idx = jnp.argmax(wk, axis=1).reshape(bq, 1)
        oh = iota == idx
        if k == 0:
            ohf0 = jnp.where(oh, 1.0, 0.0)
        if k == 5:
            ohf5 = jnp.where(oh, 1.0, 0.0)
        if k < 5:
            vv = jnp.max(wk, axis=1, keepdims=True)
            ww = jnp.exp(vv - p1)
            wk = jnp.where(oh, -ww - 2.0, wk)

    gather, pos, neg = _combine(wk, ohf0, ohf5, m)
    gath, spread = _tail_losses(q, m, gather, pos, neg)

    li = lax.broadcasted_iota(jnp.int32, (1, 128), 1)
    contrib = (jnp.where(li == 0, gath, 0.0)
               + jnp.where(li == 1, spread, 0.0))

    @pl.when(i == 0)
    def _():
        loss_ref[...] = contrib

    @pl.when(i > 0)
    def _():
        loss_ref[...] = loss_ref[...] + contrib


@functools.partial(jax.jit, static_argnames=("interpret",))
def kernel(query, m_items, interpret=False):
    q_n, d = query.shape
    mem = m_items.shape[0]
    bq = min(256, q_n)
    nqb = q_n // bq

    f32 = jnp.float32
    stat_spec = pl.BlockSpec((1, mem), lambda i: (0, 0))
    stat_shape = jax.ShapeDtypeStruct((1, mem), f32)

    bqs = min(512, q_n)
    cmax, cinv = pl.pallas_call(
        _stats_kernel,
        grid=(q_n // bqs,),
        in_specs=[pl.BlockSpec((bqs, d), lambda i: (i, 0)),
                  pl.BlockSpec((mem, d), lambda i: (0, 0))],
        out_specs=[stat_spec] * 2,
        out_shape=[stat_shape] * 2,
        compiler_params=pltpu.CompilerParams(vmem_limit_bytes=100 * 1024 * 1024),
        interpret=interpret,
    )(query, m_items)

    ssq, ssm, uq, loss = pl.pallas_call(
        _main_kernel,
        grid=(nqb,),
        in_specs=[pl.BlockSpec((bq, d), lambda i: (i, 0)),
                  pl.BlockSpec((mem, d), lambda i: (0, 0)),
                  stat_spec, stat_spec],
        out_specs=[pl.BlockSpec((bq, mem), lambda i: (i, 0)),
                   pl.BlockSpec((bq, mem), lambda i: (i, 0)),
                   pl.BlockSpec((bq, d), lambda i: (i, 0)),
                   pl.BlockSpec((1, 128), lambda i: (0, 0))],
        out_shape=[jax.ShapeDtypeStruct((q_n, mem), f32),
                   jax.ShapeDtypeStruct((q_n, mem), f32),
                   jax.ShapeDtypeStruct((q_n, d), f32),
                   jax.ShapeDtypeStruct((1, 128), f32)],
        compiler_params=pltpu.CompilerParams(vmem_limit_bytes=100 * 1024 * 1024),
        interpret=interpret,
    )(query, m_items, cmax, cinv)

    def exact_tail():
        bqe = min(128, q_n)
        loss2 = pl.pallas_call(
            _exact_kernel,
            grid=(q_n // bqe,),
            in_specs=[pl.BlockSpec((bqe, d), lambda i: (i, 0)),
                      pl.BlockSpec((mem, d), lambda i: (0, 0)),
                      pl.BlockSpec((bqe, mem), lambda i: (i, 0))],
            out_specs=pl.BlockSpec((1, 128), lambda i: (0, 0)),
            out_shape=jax.ShapeDtypeStruct((1, 128), f32),
            compiler_params=pltpu.CompilerParams(
                vmem_limit_bytes=100 * 1024 * 1024),
            interpret=interpret,
        )(query, m_items, ssm)
        return loss2[0, 0], loss2[0, 1]

    gath_sum, spread_sum = lax.cond(
        loss[0, 3] > 0.0,
        exact_tail,
        lambda: (loss[0, 1], loss[0, 2]))

    gathering_loss = gath_sum / (q_n * d)
    spreading_loss = spread_sum / q_n
    recon_loss = loss[0, 0] / (q_n * d)
    return (uq, m_items, ssq, ssm, gathering_loss, spreading_loss, recon_loss)


# speedup vs baseline: 32.3522x; 1.0009x over previous
import jax
import jax.numpy as jnp
from jax import lax
from jax.experimental import pallas as pl
from jax.experimental.pallas import tpu as pltpu


def _dot_nt(a, b):
    return lax.dot_general(a, b, (((1,), (1,)), ((), ())),
                           preferred_element_type=jnp.float32)


def _dot_nn(a, b):
    return lax.dot_general(a, b, (((1,), (0,)), ((), ())),
                           preferred_element_type=jnp.float32)


def _stats_kernel(q_ref, m_ref, cmax_ref, cinv_ref):
    i = pl.program_id(0)
    n = pl.num_programs(0)
    s = _dot_nt(q_ref[...], m_ref[...])
    bmax = jnp.max(s, axis=0, keepdims=True)

    @pl.when(i == 0)
    def _():
        cmax_ref[...] = bmax
        cinv_ref[...] = jnp.sum(jnp.exp(s - bmax), axis=0, keepdims=True)

    @pl.when(i > 0)
    def _():
        old_max = cmax_ref[...]
        new_max = jnp.maximum(old_max, bmax)
        cinv_ref[...] = (cinv_ref[...] * jnp.exp(old_max - new_max)
                         + jnp.sum(jnp.exp(s - new_max), axis=0, keepdims=True))
        cmax_ref[...] = new_max

    @pl.when(i == n - 1)
    def _():
        cinv_ref[...] = 1.0 / cinv_ref[...]


def _tail_losses(q, m, gather, pos, neg):
    gath = jnp.sum((gather - q) ** 2)
    eps = 1e-6
    d_ap = jnp.sqrt(jnp.sum((q - pos + eps) ** 2, axis=1, keepdims=True))
    d_an = jnp.sqrt(jnp.sum((q - neg + eps) ** 2, axis=1, keepdims=True))
    spread = jnp.sum(jnp.maximum(d_ap - d_an + 1.0, 0.0))
    return gath, spread


def _combine(work, ohf0, ohf5, m):
    u = jnp.maximum(-work - 2.0, 0.0)
    denom = jnp.sum(u, axis=1, keepdims=True)
    gather = _dot_nn(u, m) / denom
    pos = _dot_nn(ohf0, m)
    neg = _dot_nn(ohf5, m)
    return gather, pos, neg


def _main_kernel(q_ref, m_ref, cmax_ref, cinv_ref,
                 ssq_ref, ssm_ref, uq_ref, loss_ref):
    i = pl.program_id(0)
    bq, d = q_ref.shape
    q = q_ref[...]
    m = m_ref[...]
    s = _dot_nt(q, m)
    rmax = jnp.max(s, axis=1, keepdims=True)
    e = jnp.exp(s - rmax)
    rsum = jnp.sum(e, axis=1, keepdims=True)
    inv_rsum = 1.0 / rsum
    ssm_ref[...] = e * inv_rsum
    ssq_ref[...] = jnp.exp(s - cmax_ref[...]) * cinv_ref[...]
    uq = _dot_nn(e, m) * inv_rsum
    uq_ref[...] = uq
    recon = jnp.sum((q - uq) ** 2)

    work = ssm_ref[...]
    p1 = jnp.max(work, axis=1, keepdims=True)
    v = p1
    pos = neg = n_top6 = None
    for k in range(6):
        if k > 0:
            v = jnp.max(work, axis=1, keepdims=True)
        eq = work == v
        if k == 0:
            pos = _dot_nn(jnp.where(eq, 1.0, 0.0), m)
        if k == 5:
            eqf = jnp.where(eq, 1.0, 0.0)
            n_top6 = jnp.sum(eqf)
            neg = _dot_nn(eqf, m)
        if k < 5:
            w = jnp.exp(v - p1)
            work = jnp.where(eq, -w - 2.0, work)

    n_masked = jnp.sum(jnp.where(work < -1.5, 1.0, 0.0))
    tie = jnp.where((n_masked != 5.0 * bq) | (n_top6 != 1.0 * bq), 1.0, 0.0)

    u = jnp.maximum(-work - 2.0, 0.0)
    denom = jnp.sum(u, axis=1, keepdims=True)
    gather = _dot_nn(u, m) / denom
    gath, spread = _tail_losses(q, m, gather, pos, neg)

    li = lax.broadcasted_iota(jnp.int32, (1, 128), 1)
    contrib = (jnp.where(li == 0, recon, 0.0)
               + jnp.where(li == 1, gath, 0.0)
               + jnp.where(li == 2, spread, 0.0)
               + jnp.where(li == 3, tie, 0.0))

    @pl.when(i == 0)
    def _():
        loss_ref[...] = contrib

    @pl.when(i > 0)
    def _():
        loss_ref[...] = loss_ref[...] + contrib


def _exact_kernel(q_ref, m_ref, ssm_ref, loss_ref):
    i = pl.program_id(0)
    bq, d = q_ref.shape
    q = q_ref[...]
    m = m_ref[...]
    p = ssm_ref[...]
    p1 = jnp.max(p, axis=1, keepdims=True)
    iota = lax.broadcasted_iota(jnp.int32, p.shape, 1)
    wk = p
    ohf0 = ohf5 = None
    for k in range(6):
        idx = jnp.argmax(wk, axis=1).reshape(bq, 1)
        oh = iota == idx
        if k == 0:
            ohf0 = jnp.where(oh, 1.0, 0.0)
        if k == 5:
            ohf5 = jnp.where(oh, 1.0, 0.0)
        if k < 5:
            vv = jnp.max(wk, axis=1, keepdims=True)
            ww = jnp.exp(vv - p1)
            wk = jnp.where(oh, -ww - 2.0, wk)

    gather, pos, neg = _combine(wk, ohf0, ohf5, m)
    gath, spread = _tail_losses(q, m, gather, pos, neg)

    li = lax.broadcasted_iota(jnp.int32, (1, 128), 1)
    contrib = (jnp.where(li == 0, gath, 0.0)
               + jnp.where(li == 1, spread, 0.0))

    @pl.when(i == 0)
    def _():
        loss_ref[...] = contrib

    @pl.when(i > 0)
    def _():
        loss_ref[...] = loss_ref[...] + contrib


@jax.jit
def kernel(query, m_items):
    q_n, d = query.shape
    mem = m_items.shape[0]
    bq = min(256, q_n)
    nqb = q_n // bq

    f32 = jnp.float32
    stat_spec = pl.BlockSpec((1, mem), lambda i: (0, 0))
    stat_shape = jax.ShapeDtypeStruct((1, mem), f32)

    bqs = min(512, q_n)
    cmax, cinv = pl.pallas_call(
        _stats_kernel,
        grid=(q_n // bqs,),
        in_specs=[pl.BlockSpec((bqs, d), lambda i: (i, 0)),
                  pl.BlockSpec((mem, d), lambda i: (0, 0))],
        out_specs=[stat_spec] * 2,
        out_shape=[stat_shape] * 2,
        compiler_params=pltpu.CompilerParams(vmem_limit_bytes=100 * 1024 * 1024),
    )(query, m_items)

    ssq, ssm, uq, loss = pl.pallas_call(
        _main_kernel,
        grid=(nqb,),
        in_specs=[pl.BlockSpec((bq, d), lambda i: (i, 0)),
                  pl.BlockSpec((mem, d), lambda i: (0, 0)),
                  stat_spec, stat_spec],
        out_specs=[pl.BlockSpec((bq, mem), lambda i: (i, 0)),
                   pl.BlockSpec((bq, mem), lambda i: (i, 0)),
                   pl.BlockSpec((bq, d), lambda i: (i, 0)),
                   pl.BlockSpec((1, 128), lambda i: (0, 0))],
        out_shape=[jax.ShapeDtypeStruct((q_n, mem), f32),
                   jax.ShapeDtypeStruct((q_n, mem), f32),
                   jax.ShapeDtypeStruct((q_n, d), f32),
                   jax.ShapeDtypeStruct((1, 128), f32)],
        compiler_params=pltpu.CompilerParams(vmem_limit_bytes=100 * 1024 * 1024),
    )(query, m_items, cmax, cinv)

    def exact_tail():
        bqe = min(128, q_n)
        loss2 = pl.pallas_call(
            _exact_kernel,
            grid=(q_n // bqe,),
            in_specs=[pl.BlockSpec((bqe, d), lambda i: (i, 0)),
                      pl.BlockSpec((mem, d), lambda i: (0, 0)),
                      pl.BlockSpec((bqe, mem), lambda i: (i, 0))],
            out_specs=pl.BlockSpec((1, 128), lambda i: (0, 0)),
            out_shape=jax.ShapeDtypeStruct((1, 128), f32),
            compiler_params=pltpu.CompilerParams(
                vmem_limit_bytes=100 * 1024 * 1024),
            )(query, m_items, ssm)
        return loss2[0, 0], loss2[0, 1]

    gath_sum, spread_sum = lax.cond(
        loss[0, 3] > 0.0,
        exact_tail,
        lambda: (loss[0, 1], loss[0, 2]))

    gathering_loss = gath_sum / (q_n * d)
    spreading_loss = spread_sum / q_n
    recon_loss = loss[0, 0] / (q_n * d)
    return (uq, m_items, ssq, ssm, gathering_loss, spreading_loss, recon_loss)
